# baseline (speedup 1.0000x reference)
import functools

import jax
import jax.numpy as jnp
import numpy as np
from jax import lax
from jax.experimental import pallas as pl
from jax.experimental.pallas import tpu as pltpu
from jax.experimental.pallas import tpu_sc as plsc

E_TOTAL = 320000
D = 128
R = 31
NROWS = 5456
NPAD = 5632
TBLK = 512
CHUNK = 256
KSUB = 128


def _multiset_counts() -> np.ndarray:
    counts = np.zeros((NPAD, R), dtype=np.float32)
    for z in range(R):
        for y in range(z + 1):
            for x in range(y + 1):
                rank = (z + 2) * (z + 1) * z // 6 + (y + 1) * y // 2 + x
                counts[rank, x] += 1
                counts[rank, y] += 1
                counts[rank, z] += 1
    return counts


_COUNTS = _multiset_counts()


def _table_body(counts_ref, cb_ref, out_ref):
    out_ref[...] = jnp.dot(counts_ref[...], cb_ref[...],
                           preferred_element_type=jnp.float32)


def _sum_table(cb):
    return pl.pallas_call(
        _table_body,
        grid=(NPAD // TBLK,),
        in_specs=[
            pl.BlockSpec((TBLK, R), lambda i: (i, 0)),
            pl.BlockSpec((R, D), lambda i: (0, 0)),
        ],
        out_specs=pl.BlockSpec((TBLK, D), lambda i: (i, 0)),
        out_shape=jax.ShapeDtypeStruct((NPAD, D), jnp.float32),
    )(jnp.asarray(_COUNTS), cb)


def _make_sc_kernel(n_chunks, chunks_per_worker):
    mesh = plsc.VectorSubcoreMesh(core_axis_name="c", subcore_axis_name="s")
    n_outer = -(-chunks_per_worker // 2)

    @functools.partial(
        pl.kernel,
        mesh=mesh,
        out_type=jax.ShapeDtypeStruct((E_TOTAL, D), jnp.float32),
        scratch_types=[
            pltpu.VMEM_SHARED((NPAD, D), jnp.float32),
            pltpu.VMEM((2, CHUNK), jnp.int32),
            pltpu.VMEM((2, CHUNK), jnp.int32),
            pltpu.VMEM((2, CHUNK), jnp.int32),
            pltpu.VMEM((2, 2, KSUB), jnp.int32),
            pltpu.VMEM((2, CHUNK, D), jnp.float32),
            pltpu.SemaphoreType.DMA,
            pltpu.SemaphoreType.DMA,
            pltpu.SemaphoreType.DMA,
            pltpu.SemaphoreType.DMA,
            pltpu.SemaphoreType.DMA,
            pltpu.SemaphoreType.DMA,
        ],
    )
    def sc_kernel(i0_hbm, i1_hbm, i2_hbm, table_hbm, out_hbm,
                  tableS, i0_v, i1_v, i2_v, key_v, obuf,
                  semi0, semi1, semg0, semg1, semo0, semo1):
        cid = lax.axis_index("c")
        sid = lax.axis_index("s")
        wid = cid * 16 + sid
        semi = (semi0, semi1)
        semg = (semg0, semg1)
        semo = (semo0, semo1)

        @pl.when(sid == 0)
        def _():
            pltpu.sync_copy(table_hbm, tableS)

        plsc.subcore_barrier()

        def out_drain(b):
            pltpu.make_async_copy(
                obuf.at[b], out_hbm.at[pl.ds(0, CHUNK), :], semo[b]).wait()

        def outer_body(it2, _):
            for b in range(2):
                ch = (it2 * 2 + b) * 32 + wid

                @pl.when(ch < n_chunks)
                def _():
                    @pl.when(it2 > 0)
                    def _():
                        out_drain(b)

                    base = ch * CHUNK
                    pltpu.async_copy(
                        i0_hbm.at[pl.ds(base, CHUNK)], i0_v.at[b], semi[b])
                    pltpu.async_copy(
                        i1_hbm.at[pl.ds(base, CHUNK)], i1_v.at[b], semi[b])
                    pltpu.async_copy(
                        i2_hbm.at[pl.ds(base, CHUNK)], i2_v.at[b], semi[b])

            for b in range(2):
                ch = (it2 * 2 + b) * 32 + wid

                @pl.when(ch < n_chunks)
                def _():
                    pltpu.make_async_copy(
                        i0_hbm.at[pl.ds(0, CHUNK)], i0_v.at[b], semi[b]).wait()
                    pltpu.make_async_copy(
                        i0_hbm.at[pl.ds(0, CHUNK)], i1_v.at[b], semi[b]).wait()
                    pltpu.make_async_copy(
                        i0_hbm.at[pl.ds(0, CHUNK)], i2_v.at[b], semi[b]).wait()

                    for t in range(CHUNK // 16):
                        s = pl.ds(t * 16, 16)
                        a = jnp.minimum(i0_v[b, s], i1_v[b, s])
                        h = jnp.maximum(i0_v[b, s], i1_v[b, s])
                        z = jnp.maximum(h, i2_v[b, s])
                        m = jnp.minimum(h, i2_v[b, s])
                        y = jnp.maximum(a, m)
                        x = jnp.minimum(a, m)
                        w = ((z * (z + 1)) >> 1) * (z + 2)
                        z3 = w * jnp.int32(-1431655765)
                        y2 = (y * (y + 1)) >> 1
                        key_v[b, t * 16 // KSUB,
                              pl.ds(t * 16 % KSUB, 16)] = z3 + y2 + x

                    for j in range(CHUNK // KSUB):
                        pltpu.async_copy(
                            tableS.at[key_v.at[b, j]],
                            obuf.at[b, pl.ds(j * KSUB, KSUB), :], semg[b])

            for b in range(2):
                ch = (it2 * 2 + b) * 32 + wid

                @pl.when(ch < n_chunks)
                def _():
                    for j in range(CHUNK // KSUB):
                        pltpu.make_async_copy(
                            tableS.at[key_v.at[b, j]],
                            obuf.at[b, pl.ds(j * KSUB, KSUB), :],
                            semg[b]).wait()
                    base = ch * CHUNK
                    pltpu.async_copy(
                        obuf.at[b], out_hbm.at[pl.ds(base, CHUNK), :], semo[b])

            return ()

        lax.fori_loop(0, n_outer, outer_body, ())

        for b in range(2):
            @pl.when(b * 32 + wid < n_chunks)
            def _():
                out_drain(b)

    return sc_kernel


@jax.jit
def kernel(node2node_connection_types, codebook):
    idx = node2node_connection_types.astype(jnp.int32)
    table = _sum_table(codebook)
    n_chunks = E_TOTAL // CHUNK
    chunks_per_worker = -(-n_chunks // 32)
    sc = _make_sc_kernel(n_chunks, chunks_per_worker)
    return sc(idx[:, 0], idx[:, 1], idx[:, 2], table)
